# baseline (speedup 1.0000x reference)
import functools

import jax
import jax.numpy as jnp
from jax import lax
from jax.experimental import pallas as pl
from jax.experimental.pallas import tpu as pltpu
from jax.experimental.pallas import tpu_sc as plsc

_N = 32
_F = 3 * 512 * 512
_TOTAL = _N * _F
_COPY = (_N - 1) * _F
_NW = 32
_CHUNK = _COPY // _NW
_XCHUNK = _F // _NW

_mesh = plsc.VectorSubcoreMesh(core_axis_name="c", subcore_axis_name="s")


@functools.partial(
    pl.kernel,
    mesh=_mesh,
    out_type=jax.ShapeDtypeStruct((_TOTAL,), jnp.float32),
)
def _ring_update(x_hbm, t_hbm, out_hbm):
    wid = lax.axis_index("s") * 2 + lax.axis_index("c")
    base = pl.multiple_of(wid * _CHUNK, 8)
    pltpu.sync_copy(t_hbm.at[pl.ds(_F + base, _CHUNK)],
                    out_hbm.at[pl.ds(base, _CHUNK)])
    xb = pl.multiple_of(wid * _XCHUNK, 8)
    pltpu.sync_copy(x_hbm.at[pl.ds(xb, _XCHUNK)],
                    out_hbm.at[pl.ds(_COPY + xb, _XCHUNK)])


def kernel(x, tensors):
    out = _ring_update(x.reshape(-1), tensors.reshape(-1))
    return out.reshape(tensors.shape)


# speedup vs baseline: 12.3054x; 12.3054x over previous
import functools

import jax
import jax.numpy as jnp
from jax import lax
from jax.experimental import pallas as pl
from jax.experimental.pallas import tpu as pltpu
from jax.experimental.pallas import tpu_sc as plsc

_N = 32
_F = 3 * 512 * 512
_TOTAL = _N * _F
_COPY = (_N - 1) * _F
_NW = 32
_CHUNK = _COPY // _NW
_XCHUNK = _F // _NW
_K = 16
_B = _CHUNK // _K

_mesh = plsc.VectorSubcoreMesh(core_axis_name="c", subcore_axis_name="s")


@functools.partial(
    pl.kernel,
    mesh=_mesh,
    out_type=jax.ShapeDtypeStruct((_TOTAL,), jnp.float32),
    scratch_types=[
        pltpu.VMEM((_B,), jnp.float32),
        pltpu.VMEM((_B,), jnp.float32),
        pltpu.VMEM((_XCHUNK,), jnp.float32),
        pltpu.SemaphoreType.DMA,
        pltpu.SemaphoreType.DMA,
        pltpu.SemaphoreType.DMA,
        pltpu.SemaphoreType.DMA,
        pltpu.SemaphoreType.DMA,
    ],
)
def _ring_update(x_hbm, t_hbm, out_hbm, buf0, buf1, xbuf,
                 si0, si1, so0, so1, sx):
    wid = lax.axis_index("s") * 2 + lax.axis_index("c")
    base = pl.multiple_of(wid * _CHUNK, 8)
    xb = pl.multiple_of(wid * _XCHUNK, 8)

    bufs = (buf0, buf1)
    isems = (si0, si1)
    osems = (so0, so1)

    def in_copy(k, slot):
        return pltpu.make_async_copy(
            t_hbm.at[pl.ds(_F + base + k * _B, _B)], bufs[slot], isems[slot])

    def out_copy(k, slot):
        return pltpu.make_async_copy(
            bufs[slot], out_hbm.at[pl.ds(base + k * _B, _B)], osems[slot])

    x_in = pltpu.make_async_copy(x_hbm.at[pl.ds(xb, _XCHUNK)], xbuf, sx)
    x_in.start()
    in_copy(0, 0).start()

    for k in range(_K):
        cur = k % 2
        nxt = (k + 1) % 2
        if k + 1 < _K:
            if k >= 1:
                out_copy(k - 1, nxt).wait()
            in_copy(k + 1, nxt).start()
        in_copy(k, cur).wait()
        out_copy(k, cur).start()

    x_in.wait()
    x_out = pltpu.make_async_copy(
        xbuf, out_hbm.at[pl.ds(_COPY + xb, _XCHUNK)], sx)
    x_out.start()
    out_copy(_K - 2, (_K - 2) % 2).wait()
    out_copy(_K - 1, (_K - 1) % 2).wait()
    x_out.wait()


def kernel(x, tensors):
    out = _ring_update(x.reshape(-1), tensors.reshape(-1))
    return out.reshape(tensors.shape)
